# baseline (speedup 1.0000x reference)
import jax
import jax.numpy as jnp
from jax.experimental import pallas as pl
from jax.experimental.pallas import tpu as pltpu

MAXLEN = 8192
OUTPUT_DIM = 2048
_NDMA = 16
_ROWS = MAXLEN // _NDMA


def _copy_dma(table_ref, out_ref, sems):
    for i in range(_NDMA):
        pltpu.make_async_copy(
            table_ref.at[pl.ds(i * _ROWS, _ROWS)],
            out_ref.at[pl.ds(i * _ROWS, _ROWS)],
            sems.at[i],
        ).start()
    for i in range(_NDMA):
        pltpu.make_async_copy(
            table_ref.at[pl.ds(i * _ROWS, _ROWS)],
            out_ref.at[pl.ds(i * _ROWS, _ROWS)],
            sems.at[i],
        ).wait()


def kernel(inputs, table):
    del inputs
    out = pl.pallas_call(
        _copy_dma,
        in_specs=[pl.BlockSpec(memory_space=pl.ANY)],
        out_specs=pl.BlockSpec(memory_space=pl.ANY),
        out_shape=jax.ShapeDtypeStruct((MAXLEN, OUTPUT_DIM), table.dtype),
        scratch_shapes=[pltpu.SemaphoreType.DMA((_NDMA,))],
    )(table)
    return out[None]


# speedup vs baseline: 20.0477x; 20.0477x over previous
import jax
import jax.numpy as jnp
from jax import lax
from jax.experimental import pallas as pl
from jax.experimental.pallas import tpu as pltpu
from jax.experimental.pallas import tpu_sc as plsc

MAXLEN = 8192
OUTPUT_DIM = 2048
SPLIT = 3072

_NC = 2
_NS = 16
_NW = _NC * _NS
_ROWS_PER_W = SPLIT // _NW
_CHUNK = 16
_NCHUNKS = _ROWS_PER_W // _CHUNK

_TC_ROWS = MAXLEN - SPLIT
_TC_BLOCK = 512


def _sc_copy(table_hbm, out_hbm, buf0, buf1, in_s0, in_s1, out_s0, out_s1):
    wid = lax.axis_index("s") * _NC + lax.axis_index("c")
    base = wid * _ROWS_PER_W
    bufs = (buf0, buf1)
    in_sems = (in_s0, in_s1)
    out_sems = (out_s0, out_s1)
    for i in range(_NCHUNKS):
        b = i % 2
        lo = base + i * _CHUNK
        if i >= 2:
            pltpu.make_async_copy(bufs[b], out_hbm.at[pl.ds(lo - 2 * _CHUNK, _CHUNK)],
                                  out_sems[b]).wait()
        cin = pltpu.make_async_copy(table_hbm.at[pl.ds(lo, _CHUNK)], bufs[b],
                                    in_sems[b])
        cin.start()
        cin.wait()
        pltpu.make_async_copy(bufs[b], out_hbm.at[pl.ds(lo, _CHUNK)],
                              out_sems[b]).start()
    for i in range(_NCHUNKS - 2, _NCHUNKS):
        b = i % 2
        lo = base + i * _CHUNK
        pltpu.make_async_copy(bufs[b], out_hbm.at[pl.ds(lo, _CHUNK)],
                              out_sems[b]).wait()


def _tc_copy(table_ref, out_ref):
    out_ref[...] = table_ref[...]


def kernel(inputs, table):
    del inputs
    mesh = plsc.VectorSubcoreMesh(core_axis_name="c", subcore_axis_name="s")
    sc_part = pl.kernel(
        _sc_copy,
        mesh=mesh,
        out_type=jax.ShapeDtypeStruct((SPLIT, OUTPUT_DIM), table.dtype),
        scratch_types=[
            pltpu.VMEM((_CHUNK, OUTPUT_DIM), jnp.float32),
            pltpu.VMEM((_CHUNK, OUTPUT_DIM), jnp.float32),
            pltpu.SemaphoreType.DMA,
            pltpu.SemaphoreType.DMA,
            pltpu.SemaphoreType.DMA,
            pltpu.SemaphoreType.DMA,
        ],
    )(table)
    tc_part = pl.pallas_call(
        _tc_copy,
        grid=(_TC_ROWS // _TC_BLOCK,),
        in_specs=[pl.BlockSpec((_TC_BLOCK, OUTPUT_DIM),
                               lambda i: (i + SPLIT // _TC_BLOCK, 0))],
        out_specs=pl.BlockSpec((_TC_BLOCK, OUTPUT_DIM), lambda i: (i, 0)),
        out_shape=jax.ShapeDtypeStruct((_TC_ROWS, OUTPUT_DIM), table.dtype),
    )(table)
    return jnp.concatenate([sc_part, tc_part], axis=0)[None]


# speedup vs baseline: 49.7891x; 2.4835x over previous
import jax
import jax.numpy as jnp
from jax.experimental import pallas as pl
from jax.experimental.pallas import tpu as pltpu

MAXLEN = 8192
OUTPUT_DIM = 2048
_CHUNK = 512
_NCHUNK = MAXLEN // _CHUNK
_NBUF = 8


def _copy_ring(table_ref, out_ref, bufs, in_sems, out_sems):
    def cin(i):
        return pltpu.make_async_copy(
            table_ref.at[pl.ds(i * _CHUNK, _CHUNK)], bufs.at[i % _NBUF],
            in_sems.at[i % _NBUF])

    def cout(i):
        return pltpu.make_async_copy(
            bufs.at[i % _NBUF], out_ref.at[pl.ds(i * _CHUNK, _CHUNK)],
            out_sems.at[i % _NBUF])

    for i in range(_NBUF):
        cin(i).start()
    for i in range(_NCHUNK):
        cin(i).wait()
        cout(i).start()
        if i + _NBUF < _NCHUNK:
            cout(i).wait()
            cin(i + _NBUF).start()
    for i in range(_NCHUNK - _NBUF, _NCHUNK):
        cout(i).wait()


def kernel(inputs, table):
    del inputs
    out = pl.pallas_call(
        _copy_ring,
        in_specs=[pl.BlockSpec(memory_space=pl.ANY)],
        out_specs=pl.BlockSpec(memory_space=pl.ANY),
        out_shape=jax.ShapeDtypeStruct((MAXLEN, OUTPUT_DIM), table.dtype),
        scratch_shapes=[
            pltpu.VMEM((_NBUF, _CHUNK, OUTPUT_DIM), jnp.float32),
            pltpu.SemaphoreType.DMA((_NBUF,)),
            pltpu.SemaphoreType.DMA((_NBUF,)),
        ],
    )(table)
    return out[None]
